# baseline (speedup 1.0000x reference)
import functools

import jax
import jax.numpy as jnp
from jax import lax
from jax.experimental import pallas as pl
from jax.experimental.pallas import tpu as pltpu
from jax.experimental.pallas import tpu_sc as plsc

VOCAB = 1_000_000
EMBED_DIM = 64
BATCH = 16384
LANES = 128

_info = plsc.get_sparse_core_info()
_NC, _NS = _info.num_cores, _info.num_subcores
_NW = _NC * _NS

_TC_TOTAL = 7812
_TC_PER_W = 245
_CH_TC = 5
_NCHUNK = _TC_PER_W // _CH_TC
_CH_W = _CH_TC * LANES
_LIST_CAP = 1024
_MATCH_CAP = 32
_SCRATCH_ROWS = BATCH + 8

_mesh = plsc.VectorSubcoreMesh(core_axis_name="c", subcore_axis_name="s")


@functools.partial(
    pl.kernel,
    mesh=_mesh,
    out_type=jax.ShapeDtypeStruct((_SCRATCH_ROWS, LANES), jnp.float32),
    scratch_types=[
        pltpu.VMEM((BATCH,), jnp.int32),
        pltpu.VMEM((_LIST_CAP,), jnp.int32),
        pltpu.VMEM((_LIST_CAP,), jnp.int32),
        pltpu.VMEM((2, EMBED_DIM, _CH_W), jnp.float32),
        pltpu.VMEM((2, _MATCH_CAP, LANES), jnp.float32),
        pltpu.VMEM((2, _MATCH_CAP), jnp.int32),
        pltpu.VMEM((_MATCH_CAP,), jnp.int32),
        pltpu.VMEM((_MATCH_CAP,), jnp.int32),
        pltpu.SemaphoreType.DMA,
        pltpu.SemaphoreType.DMA,
    ],
    compiler_params=pltpu.CompilerParams(needs_layout_passes=False),
)
def _scan_kernel(idx_hbm, table_t_hbm, scratch_hbm, idx_all, lidx, lpos,
                 chunks_v, stage_v, stagep_v, midx, mpos, sem, sem_out):
    wid = lax.axis_index("s") * _NC + lax.axis_index("c")
    tc_start = jnp.minimum(wid * _TC_PER_W, _TC_TOTAL - _TC_PER_W)
    row_lo = tc_start * LANES
    row_hi = row_lo + _TC_PER_W * LANES
    row_hi = jnp.where(wid == _NW - 1, jnp.int32(1 << 30), row_hi)

    pltpu.sync_copy(idx_hbm, idx_all)

    iota = lax.iota(jnp.int32, 16)
    row16 = [iota + 16 * g for g in range(EMBED_DIM // 16)]

    def filt(m, ptr):
        iv = idx_all[pl.ds(m * 16, 16)]
        pv = iota + m * 16
        msk = (iv >= row_lo) & (iv < row_hi)
        mi = msk.astype(jnp.int32)
        excl = plsc.cumsum(mi) - mi
        dest = ptr + excl
        plsc.store_scatter(lidx, [dest], iv, mask=msk)
        plsc.store_scatter(lpos, [dest], pv, mask=msk)
        return ptr + plsc.all_reduce_population_count(msk)

    ptr = lax.fori_loop(0, BATCH // 16, filt, jnp.zeros((16,), jnp.int32),
                        unroll=False)
    n_list = ptr[0]
    ng = (n_list + 15) // 16

    def chunk_col0(k):
        c = jnp.where(k == _NCHUNK, VOCAB + 64 - _CH_W, (tc_start + k * _CH_TC) * LANES)
        return pl.multiple_of(c, LANES)

    def issue(bank, k):
        pltpu.async_copy(
            table_t_hbm.at[:, pl.ds(chunk_col0(k), _CH_W)],
            chunks_v.at[bank],
            sem,
        )

    def process(bank, k):
        pltpu.make_async_copy(
            table_t_hbm.at[:, pl.ds(0, _CH_W)], chunks_v.at[bank], sem
        ).wait()
        lo = chunk_col0(k)
        hi = lo + _CH_W
        def match(j, mp):
            iv = lidx[pl.ds(j * 16, 16)]
            pv = lpos[pl.ds(j * 16, 16)]
            valid = (iota + j * 16) < n_list
            msk = (iv >= lo) & (iv < hi) & valid
            mi = msk.astype(jnp.int32)
            excl = plsc.cumsum(mi) - mi
            dest = mp + excl
            plsc.store_scatter(midx, [dest], iv, mask=msk)
            plsc.store_scatter(mpos, [dest], pv, mask=msk)
            return mp + plsc.all_reduce_population_count(msk)

        mp = lax.fori_loop(0, ng, match, jnp.zeros((16,), jnp.int32),
                           unroll=False)
        n_match = mp[0]
        dust = jnp.broadcast_to(jnp.int32(BATCH), (16,))
        plsc.store_scatter(stagep_v.at[bank], [iota], dust)
        plsc.store_scatter(stagep_v.at[bank], [iota + 16], dust)

        def extract(e, _):
            sv = jnp.broadcast_to(e, (16,))
            v = plsc.load_gather(midx, [sv])[0]
            p = plsc.load_gather(mpos, [sv])[0]
            lane = jnp.broadcast_to(v - lo, (16,))
            plsc.store_scatter(stagep_v.at[bank], [sv], jnp.broadcast_to(p, (16,)),
                               mask=iota < 1)
            for g, r16 in enumerate(row16):
                vals = plsc.load_gather(chunks_v.at[bank], [r16, lane])
                plsc.store_scatter(stage_v.at[bank], [sv, r16], vals)
            return ()

        lax.fori_loop(0, n_match, extract, (), unroll=False)
        pltpu.async_copy(
            stage_v.at[bank], scratch_hbm.at[stagep_v.at[bank]], sem_out
        )

    def drain_out(bank):
        pltpu.make_async_copy(
            stage_v.at[bank], scratch_hbm.at[stagep_v.at[bank]], sem_out
        ).wait()

    issue(0, jnp.int32(0))

    def pair(p, _):
        issue(1, 2 * p + 1)
        @pl.when(p > 0)
        def _():
            drain_out(0)
        process(0, 2 * p)
        @pl.when(p < (_NCHUNK + 1) // 2 - 1)
        def _():
            issue(0, 2 * p + 2)
        @pl.when(p > 0)
        def _():
            drain_out(1)
        process(1, 2 * p + 1)
        return ()

    lax.fori_loop(0, (_NCHUNK + 1) // 2, pair, (), unroll=False)
    drain_out(0)
    drain_out(1)


def kernel(go_terms, min_embedding):
    idx = go_terms.astype(jnp.int32)
    scratch = _scan_kernel(idx, min_embedding.T)
    return scratch[:BATCH, :EMBED_DIM]


# speedup vs baseline: 8.6147x; 8.6147x over previous
import functools

import jax
import jax.numpy as jnp
from jax import lax
from jax.experimental import pallas as pl
from jax.experimental.pallas import tpu as pltpu
from jax.experimental.pallas import tpu_sc as plsc

VOCAB = 1_000_000
EMBED_DIM = 64
BATCH = 16384
LANES = 128

_info = plsc.get_sparse_core_info()
_NC, _NS = _info.num_cores, _info.num_subcores
_NW = _NC * _NS
_B_PER_W = BATCH // _NW
_K = 8
_NBURST = _B_PER_W // _K

_mesh = plsc.VectorSubcoreMesh(core_axis_name="c", subcore_axis_name="s")


@functools.partial(
    pl.kernel,
    mesh=_mesh,
    out_type=jax.ShapeDtypeStruct((EMBED_DIM, BATCH), jnp.float32),
    scratch_types=[
        pltpu.VMEM((_B_PER_W + 8,), jnp.int32),
        pltpu.VMEM((_K, EMBED_DIM, LANES), jnp.float32),
        pltpu.VMEM((EMBED_DIM, _B_PER_W), jnp.float32),
        pltpu.SemaphoreType.DMA,
    ],
    compiler_params=pltpu.CompilerParams(needs_layout_passes=False),
)
def _gather_kernel(idx_hbm, table_t_hbm, out_t_hbm, idx_v, cols_v, out_v, sem):
    wid = lax.axis_index("s") * _NC + lax.axis_index("c")
    base = wid * _B_PER_W
    pltpu.sync_copy(idx_hbm.at[pl.ds(base, _B_PER_W)], idx_v.at[pl.ds(0, _B_PER_W)])

    row16 = [lax.iota(jnp.int32, 16) + 16 * g for g in range(EMBED_DIM // 16)]
    half_slots = _K // 2

    def issue(bank, idx16, off):
        for b in range(half_slots):
            v = idx16[off + b]
            col0 = pl.multiple_of((v // LANES) * LANES, LANES)
            pltpu.async_copy(
                table_t_hbm.at[:, pl.ds(col0, LANES)],
                cols_v.at[bank * half_slots + b],
                sem,
            )

    def drain_extract(bank, idx16, off, ibase):
        pltpu.make_async_copy(
            table_t_hbm.at[:, pl.ds(0, half_slots * LANES)],
            cols_v.at[pl.ds(bank * half_slots, half_slots)],
            sem,
        ).wait()
        for b in range(half_slots):
            v = idx16[off + b]
            lane = jnp.broadcast_to(v % LANES, (16,))
            ivec = jnp.broadcast_to(ibase + b, (16,))
            for r16 in row16:
                vals = plsc.load_gather(cols_v.at[bank * half_slots + b], [r16, lane])
                plsc.store_scatter(out_v, [r16, ivec], vals)

    def group(g, idx_prev):
        idx16 = idx_v[pl.ds(g * 8, 16)]

        @pl.when(g > 0)
        def _():
            drain_extract(0, idx_prev, 0, (g - 1) * 8)

        issue(0, idx16, 0)

        @pl.when(g > 0)
        def _():
            drain_extract(1, idx_prev, half_slots, (g - 1) * 8 + half_slots)

        issue(1, idx16, half_slots)
        return idx16

    idx_last = lax.fori_loop(
        0, _B_PER_W // 8, group, jnp.zeros((16,), jnp.int32), unroll=False
    )
    drain_extract(0, idx_last, 0, _B_PER_W - 8)
    drain_extract(1, idx_last, half_slots, _B_PER_W - half_slots)

    pltpu.sync_copy(out_v, out_t_hbm.at[:, pl.ds(base, _B_PER_W)])


def kernel(go_terms, min_embedding):
    idx = go_terms.astype(jnp.int32)
    out_t = _gather_kernel(idx, min_embedding.T)
    return out_t.T
